# initial kernel scaffold (unmeasured)
import jax
import jax.numpy as jnp
from jax import lax
from jax.experimental import pallas as pl
from jax.experimental.pallas import tpu as pltpu

_DeviceIdType = getattr(pl, "DeviceIdType", None) or pltpu.DeviceIdType
_sem_signal = getattr(pl, "semaphore_signal", None) or pltpu.semaphore_signal
_sem_wait = getattr(pl, "semaphore_wait", None) or pltpu.semaphore_wait

D = 2048
F = 8192
D_HALF = D // 2
F_HALF = F // 2


def kernel(x, dy):
    my_x = lax.axis_index("x")

    x16 = x.astype(jnp.bfloat16)
    dy_half = lax.dynamic_slice_in_dim(dy, my_x * F_HALF, F_HALF, axis=1)
    dy16 = dy_half.astype(jnp.bfloat16)
    p = lax.dot_general(
        x16, dy16,
        dimension_numbers=(((0,), (0,)), ((), ())),
        preferred_element_type=jnp.float32,
    ).astype(jnp.bfloat16)

    def body(p_ref, out_ref, rs_recv, ag_src, ag_recv,
             rs_send_sem, rs_recv_sem, ag_send_sem, ag_recv_sem):
        mx = lax.axis_index("x")
        my = lax.axis_index("y")
        x_nbr = (1 - mx, my)
        y_nbr = (mx, 1 - my)

        barrier = pltpu.get_barrier_semaphore()
        for nbr in (x_nbr, y_nbr):
            _sem_signal(barrier, inc=1, device_id=nbr,
                        device_id_type=_DeviceIdType.MESH)
        _sem_wait(barrier, 2)

        rs = pltpu.make_async_remote_copy(
            src_ref=p_ref.at[pl.ds((1 - my) * D_HALF, D_HALF), :],
            dst_ref=rs_recv,
            send_sem=rs_send_sem,
            recv_sem=rs_recv_sem,
            device_id=y_nbr,
            device_id_type=_DeviceIdType.MESH,
        )
        rs.start()
        rs.wait()

        mine = p_ref[pl.ds(my * D_HALF, D_HALF), :]
        red = mine.astype(jnp.float32) + rs_recv[...].astype(jnp.float32)
        out_ref[:, pl.ds(mx * F_HALF, F_HALF)] = red
        ag_src[...] = red.astype(jnp.bfloat16)

        ag = pltpu.make_async_remote_copy(
            src_ref=ag_src,
            dst_ref=ag_recv,
            send_sem=ag_send_sem,
            recv_sem=ag_recv_sem,
            device_id=x_nbr,
            device_id_type=_DeviceIdType.MESH,
        )
        ag.start()
        ag.wait()
        out_ref[:, pl.ds((1 - mx) * F_HALF, F_HALF)] = (
            ag_recv[...].astype(jnp.float32)
        )

    return pl.pallas_call(
        body,
        out_shape=jax.ShapeDtypeStruct((D_HALF, F), jnp.float32),
        in_specs=[pl.BlockSpec(memory_space=pltpu.VMEM)],
        out_specs=pl.BlockSpec(memory_space=pltpu.VMEM),
        scratch_shapes=[
            pltpu.VMEM((D_HALF, F_HALF), jnp.bfloat16),
            pltpu.VMEM((D_HALF, F_HALF), jnp.bfloat16),
            pltpu.VMEM((D_HALF, F_HALF), jnp.bfloat16),
            pltpu.SemaphoreType.DMA,
            pltpu.SemaphoreType.DMA,
            pltpu.SemaphoreType.DMA,
            pltpu.SemaphoreType.DMA,
        ],
        compiler_params=pltpu.CompilerParams(collective_id=0),
    )(p)


# baseline (device time: 266962 ns/iter reference)
import jax
import jax.numpy as jnp
from jax import lax
from jax.experimental import pallas as pl
from jax.experimental.pallas import tpu as pltpu

_DeviceIdType = getattr(pl, "DeviceIdType", None) or pltpu.DeviceIdType
_sem_signal = getattr(pl, "semaphore_signal", None) or pltpu.semaphore_signal
_sem_wait = getattr(pl, "semaphore_wait", None) or pltpu.semaphore_wait

D = 2048
F = 8192
D_HALF = D // 2
F_HALF = F // 2
FC = 1024


def kernel(x, dy):
    my_x = lax.axis_index("x")

    x16 = x.astype(jnp.bfloat16)
    dy_half = lax.dynamic_slice_in_dim(dy, my_x * F_HALF, F_HALF, axis=1)
    dy16 = dy_half.astype(jnp.bfloat16)
    p = lax.dot_general(
        x16, dy16,
        dimension_numbers=(((0,), (0,)), ((), ())),
        preferred_element_type=jnp.float32,
    ).astype(jnp.bfloat16)

    def body(p_ref, out_ref, rs_recv, ag_recv,
             rs_send_sem, rs_recv_sem, ag_send_sem, ag_recv_sem):
        mx = lax.axis_index("x")
        my = lax.axis_index("y")
        x_nbr = (1 - mx, my)
        y_nbr = (mx, 1 - my)

        barrier = pltpu.get_barrier_semaphore()
        for nbr in (x_nbr, y_nbr):
            _sem_signal(barrier, inc=1, device_id=nbr,
                        device_id_type=_DeviceIdType.MESH)
        _sem_wait(barrier, 2)

        rs = pltpu.make_async_remote_copy(
            src_ref=p_ref.at[pl.ds((1 - my) * D_HALF, D_HALF), :],
            dst_ref=rs_recv,
            send_sem=rs_send_sem,
            recv_sem=rs_recv_sem,
            device_id=y_nbr,
            device_id_type=_DeviceIdType.MESH,
        )
        rs.start()
        rs.wait()

        for c in range(F_HALF // FC):
            col = pl.ds(c * FC, FC)
            mine = p_ref[pl.ds(my * D_HALF, D_HALF), col].astype(jnp.float32)
            red = (mine + rs_recv[:, col].astype(jnp.float32)).astype(
                jnp.bfloat16
            )
            rs_recv[:, col] = red
        out_ref[:, pl.ds(mx * F_HALF, F_HALF)] = rs_recv[...]

        ag = pltpu.make_async_remote_copy(
            src_ref=rs_recv,
            dst_ref=ag_recv,
            send_sem=ag_send_sem,
            recv_sem=ag_recv_sem,
            device_id=x_nbr,
            device_id_type=_DeviceIdType.MESH,
        )
        ag.start()
        ag.wait()
        out_ref[:, pl.ds((1 - mx) * F_HALF, F_HALF)] = ag_recv[...]

    return pl.pallas_call(
        body,
        out_shape=jax.ShapeDtypeStruct((D_HALF, F), jnp.bfloat16),
        in_specs=[pl.BlockSpec(memory_space=pltpu.VMEM)],
        out_specs=pl.BlockSpec(memory_space=pltpu.VMEM),
        scratch_shapes=[
            pltpu.VMEM((D_HALF, F_HALF), jnp.bfloat16),
            pltpu.VMEM((D_HALF, F_HALF), jnp.bfloat16),
            pltpu.SemaphoreType.DMA,
            pltpu.SemaphoreType.DMA,
            pltpu.SemaphoreType.DMA,
            pltpu.SemaphoreType.DMA,
        ],
        compiler_params=pltpu.CompilerParams(collective_id=0),
    )(p)


# device time: 151502 ns/iter; 1.7621x vs baseline; 1.7621x over previous
import jax
import jax.numpy as jnp
from jax import lax
from jax.experimental import pallas as pl
from jax.experimental.pallas import tpu as pltpu

_DeviceIdType = getattr(pl, "DeviceIdType", None) or pltpu.DeviceIdType
_sem_signal = getattr(pl, "semaphore_signal", None) or pltpu.semaphore_signal
_sem_wait = getattr(pl, "semaphore_wait", None) or pltpu.semaphore_wait

M = 2048
D = 2048
F = 8192
D_HALF = D // 2
F_HALF = F // 2
NC = 8
FC = F_HALF // NC
DT = 512


def kernel(x, dy):
    xt16 = x.astype(jnp.bfloat16).T

    def body(xt_ref, dy_ref, out_ref, dy_vmem, p_buf, rs_recv,
             dy_sems, rs_send_sems, rs_recv_sems, ag_send_sems, ag_recv_sems):
        mx = lax.axis_index("x")
        my = lax.axis_index("y")
        x_nbr = (1 - mx, my)
        y_nbr = (mx, 1 - my)

        def dy_fetch(c):
            cp = pltpu.make_async_copy(
                dy_ref.at[:, pl.ds(mx * F_HALF + c * FC, FC)],
                dy_vmem.at[c % 2],
                dy_sems.at[c % 2],
            )
            cp.start()
            return cp

        dy_cp = dy_fetch(0)

        barrier = pltpu.get_barrier_semaphore()
        for nbr in (x_nbr, y_nbr):
            _sem_signal(barrier, inc=1, device_id=nbr,
                        device_id_type=_DeviceIdType.MESH)
        _sem_wait(barrier, 2)

        rs_ops = [None] * NC
        ag_ops = [None] * NC

        def finish(c):
            rs_ops[c].wait_recv()
            col = pl.ds(mx * F_HALF + c * FC, FC)
            out_ref[:, col] = (
                p_buf[c % 2, pl.ds(my * D_HALF, D_HALF), :] + rs_recv[c]
            )
            ag = pltpu.make_async_remote_copy(
                src_ref=out_ref.at[:, col],
                dst_ref=out_ref.at[:, col],
                send_sem=ag_send_sems.at[c],
                recv_sem=ag_recv_sems.at[c],
                device_id=x_nbr,
                device_id_type=_DeviceIdType.MESH,
            )
            ag.start()
            ag_ops[c] = ag

        for c in range(NC):
            next_cp = dy_fetch(c + 1) if c + 1 < NC else None
            dy_cp.wait()
            dy_cp = next_cp
            if c >= 2:
                rs_ops[c - 2].wait_send()
            b = dy_vmem[c % 2].astype(jnp.bfloat16)
            for dt in range(D // DT):
                p_buf[c % 2, pl.ds(dt * DT, DT), :] = lax.dot_general(
                    xt_ref[pl.ds(dt * DT, DT), :], b,
                    dimension_numbers=(((1,), (0,)), ((), ())),
                    preferred_element_type=jnp.float32,
                ).astype(jnp.bfloat16)
            rs = pltpu.make_async_remote_copy(
                src_ref=p_buf.at[c % 2, pl.ds((1 - my) * D_HALF, D_HALF), :],
                dst_ref=rs_recv.at[c],
                send_sem=rs_send_sems.at[c],
                recv_sem=rs_recv_sems.at[c],
                device_id=y_nbr,
                device_id_type=_DeviceIdType.MESH,
            )
            rs.start()
            rs_ops[c] = rs
            if c >= 1:
                finish(c - 1)
        finish(NC - 1)

        rs_ops[NC - 2].wait_send()
        rs_ops[NC - 1].wait_send()
        for c in range(NC):
            ag_ops[c].wait_send()
            ag_ops[c].wait_recv()

    return pl.pallas_call(
        body,
        out_shape=jax.ShapeDtypeStruct((D_HALF, F), jnp.bfloat16),
        in_specs=[
            pl.BlockSpec(memory_space=pltpu.VMEM),
            pl.BlockSpec(memory_space=pl.ANY),
        ],
        out_specs=pl.BlockSpec(memory_space=pltpu.VMEM),
        scratch_shapes=[
            pltpu.VMEM((2, M, FC), jnp.float32),
            pltpu.VMEM((2, D, FC), jnp.bfloat16),
            pltpu.VMEM((NC, D_HALF, FC), jnp.bfloat16),
            pltpu.SemaphoreType.DMA((2,)),
            pltpu.SemaphoreType.DMA((NC,)),
            pltpu.SemaphoreType.DMA((NC,)),
            pltpu.SemaphoreType.DMA((NC,)),
            pltpu.SemaphoreType.DMA((NC,)),
        ],
        compiler_params=pltpu.CompilerParams(
            collective_id=0,
            vmem_limit_bytes=60 * 1024 * 1024,
        ),
    )(xt16, dy)


# device time: 144661 ns/iter; 1.8454x vs baseline; 1.0473x over previous
import jax
import jax.numpy as jnp
from jax import lax
from jax.experimental import pallas as pl
from jax.experimental.pallas import tpu as pltpu

_DeviceIdType = getattr(pl, "DeviceIdType", None) or pltpu.DeviceIdType
_sem_signal = getattr(pl, "semaphore_signal", None) or pltpu.semaphore_signal
_sem_wait = getattr(pl, "semaphore_wait", None) or pltpu.semaphore_wait

M = 2048
D = 2048
F = 8192
D_HALF = D // 2
F_HALF = F // 2
NC = 8
FC = F_HALF // NC
DT = 512


def kernel(x, dy):
    x16 = x.astype(jnp.bfloat16)

    def body(xt_ref, dy_ref, out_ref, dy_vmem, p_buf, rs_recv,
             dy_sems, rs_send_sems, rs_recv_sems, ag_send_sems, ag_recv_sems):
        mx = lax.axis_index("x")
        my = lax.axis_index("y")
        x_nbr = (1 - mx, my)
        y_nbr = (mx, 1 - my)

        def dy_fetch(c):
            cp = pltpu.make_async_copy(
                dy_ref.at[:, pl.ds(mx * F_HALF + c * FC, FC)],
                dy_vmem.at[c % 2],
                dy_sems.at[c % 2],
            )
            cp.start()
            return cp

        dy_cp = dy_fetch(0)

        barrier = pltpu.get_barrier_semaphore()
        for nbr in (x_nbr, y_nbr):
            _sem_signal(barrier, inc=1, device_id=nbr,
                        device_id_type=_DeviceIdType.MESH)
        _sem_wait(barrier, 2)

        rs_ops = [None] * NC
        ag_ops = [None] * NC

        def finish(c):
            rs_ops[c].wait_recv()
            col = pl.ds(mx * F_HALF + c * FC, FC)
            out_ref[:, col] = (
                p_buf[c % 2, pl.ds(my * D_HALF, D_HALF), :] + rs_recv[c]
            )
            ag = pltpu.make_async_remote_copy(
                src_ref=out_ref.at[:, col],
                dst_ref=out_ref.at[:, col],
                send_sem=ag_send_sems.at[c],
                recv_sem=ag_recv_sems.at[c],
                device_id=x_nbr,
                device_id_type=_DeviceIdType.MESH,
            )
            ag.start()
            ag_ops[c] = ag

        for c in range(NC):
            next_cp = dy_fetch(c + 1) if c + 1 < NC else None
            dy_cp.wait()
            dy_cp = next_cp
            if c >= 2:
                rs_ops[c - 2].wait_send()
            b = dy_vmem[c % 2].astype(jnp.bfloat16)

            def mm_rows(row0):
                for i in range(D_HALF // DT):
                    rows = pl.ds(row0 + i * DT, DT)
                    p_buf[c % 2, rows, :] = lax.dot_general(
                        xt_ref[:, rows], b,
                        dimension_numbers=(((0,), (0,)), ((), ())),
                        preferred_element_type=jnp.float32,
                    ).astype(jnp.bfloat16)

            mm_rows((1 - my) * D_HALF)
            rs = pltpu.make_async_remote_copy(
                src_ref=p_buf.at[c % 2, pl.ds((1 - my) * D_HALF, D_HALF), :],
                dst_ref=rs_recv.at[c],
                send_sem=rs_send_sems.at[c],
                recv_sem=rs_recv_sems.at[c],
                device_id=y_nbr,
                device_id_type=_DeviceIdType.MESH,
            )
            rs.start()
            rs_ops[c] = rs
            mm_rows(my * D_HALF)
            if c >= 1:
                finish(c - 1)
        finish(NC - 1)

        rs_ops[NC - 2].wait_send()
        rs_ops[NC - 1].wait_send()
        for c in range(NC):
            ag_ops[c].wait_send()
            ag_ops[c].wait_recv()

    return pl.pallas_call(
        body,
        out_shape=jax.ShapeDtypeStruct((D_HALF, F), jnp.bfloat16),
        in_specs=[
            pl.BlockSpec(memory_space=pltpu.VMEM),
            pl.BlockSpec(memory_space=pl.ANY),
        ],
        out_specs=pl.BlockSpec(memory_space=pltpu.VMEM),
        scratch_shapes=[
            pltpu.VMEM((2, M, FC), jnp.float32),
            pltpu.VMEM((2, D, FC), jnp.bfloat16),
            pltpu.VMEM((NC, D_HALF, FC), jnp.bfloat16),
            pltpu.SemaphoreType.DMA((2,)),
            pltpu.SemaphoreType.DMA((NC,)),
            pltpu.SemaphoreType.DMA((NC,)),
            pltpu.SemaphoreType.DMA((NC,)),
            pltpu.SemaphoreType.DMA((NC,)),
        ],
        compiler_params=pltpu.CompilerParams(
            collective_id=0,
            vmem_limit_bytes=60 * 1024 * 1024,
        ),
    )(x16, dy)


# device time: 130090 ns/iter; 2.0521x vs baseline; 1.1120x over previous
import jax
import jax.numpy as jnp
from jax import lax
from jax.experimental import pallas as pl
from jax.experimental.pallas import tpu as pltpu

_DeviceIdType = getattr(pl, "DeviceIdType", None) or pltpu.DeviceIdType
_sem_signal = getattr(pl, "semaphore_signal", None) or pltpu.semaphore_signal
_sem_wait = getattr(pl, "semaphore_wait", None) or pltpu.semaphore_wait

M = 2048
D = 2048
F = 8192
D_HALF = D // 2
F_HALF = F // 2
NC = 16
FC = F_HALF // NC
XT = 512
NPB = 4


def kernel(x, dy):

    def body(x_ref, dy_ref, out_ref, x16, x_stage, dy_vmem, p_buf,
             red_stage, rs_recv,
             x_sems, dy_sems, out_sems,
             rs_send_sems, rs_recv_sems, ag_send_sems, ag_recv_sems):
        mx = lax.axis_index("x")
        my = lax.axis_index("y")
        x_nbr = (1 - mx, my)
        y_nbr = (mx, 1 - my)

        barrier = pltpu.get_barrier_semaphore()
        for nbr in (x_nbr, y_nbr):
            _sem_signal(barrier, inc=1, device_id=nbr,
                        device_id_type=_DeviceIdType.MESH)

        def dy_fetch(c):
            cp = pltpu.make_async_copy(
                dy_ref.at[:, pl.ds(mx * F_HALF + c * FC, FC)],
                dy_vmem.at[c % 3],
                dy_sems.at[c % 3],
            )
            cp.start()
            return cp

        dy_cps = [None] * NC
        dy_cps[0] = dy_fetch(0)
        dy_cps[1] = dy_fetch(1)

        n_xt = M // XT
        x_cps = [None] * n_xt
        for i in range(n_xt):
            x_cps[i] = pltpu.make_async_copy(
                x_ref.at[pl.ds(i * XT, XT), :], x_stage.at[i % 2],
                x_sems.at[i % 2],
            )
            x_cps[i].start()
            if i >= 1:
                x_cps[i - 1].wait()
                x16[pl.ds((i - 1) * XT, XT), :] = (
                    x_stage[(i - 1) % 2].astype(jnp.bfloat16)
                )
        x_cps[n_xt - 1].wait()
        x16[pl.ds((n_xt - 1) * XT, XT), :] = (
            x_stage[(n_xt - 1) % 2].astype(jnp.bfloat16)
        )

        rs_ops = [None] * NC
        ag_ops = [None] * NC
        out_cps = [None] * NC

        def finish(c):
            slot = c % NPB
            if c >= NPB:
                out_cps[c - NPB].wait()
                ag_ops[c - NPB].wait_send()
            rs_ops[c].wait_recv()
            red_stage[slot] = (
                p_buf[c % NPB, pl.ds(my * D_HALF, D_HALF), :] + rs_recv[c]
            )
            col = pl.ds(mx * F_HALF + c * FC, FC)
            cp = pltpu.make_async_copy(
                red_stage.at[slot], out_ref.at[:, col], out_sems.at[slot]
            )
            cp.start()
            out_cps[c] = cp
            ag = pltpu.make_async_remote_copy(
                src_ref=red_stage.at[slot],
                dst_ref=out_ref.at[:, col],
                send_sem=ag_send_sems.at[c],
                recv_sem=ag_recv_sems.at[c],
                device_id=x_nbr,
                device_id_type=_DeviceIdType.MESH,
            )
            ag.start()
            ag_ops[c] = ag

        for c in range(NC):
            dy_cps[c].wait()
            b = dy_vmem[c % 3].astype(jnp.bfloat16)
            if c + 2 < NC:
                dy_cps[c + 2] = dy_fetch(c + 2)
            if c >= NPB:
                rs_ops[c - NPB].wait_send()

            def mm_half(row0):
                rows = pl.ds(row0, D_HALF)
                p_buf[c % NPB, rows, :] = lax.dot_general(
                    x16[:, rows], b,
                    dimension_numbers=(((0,), (0,)), ((), ())),
                    preferred_element_type=jnp.float32,
                ).astype(jnp.bfloat16)

            mm_half((1 - my) * D_HALF)
            if c == 0:
                _sem_wait(barrier, 2)
            rs = pltpu.make_async_remote_copy(
                src_ref=p_buf.at[c % NPB, pl.ds((1 - my) * D_HALF, D_HALF), :],
                dst_ref=rs_recv.at[c],
                send_sem=rs_send_sems.at[c],
                recv_sem=rs_recv_sems.at[c],
                device_id=y_nbr,
                device_id_type=_DeviceIdType.MESH,
            )
            rs.start()
            rs_ops[c] = rs
            mm_half(my * D_HALF)
            if c >= 2:
                finish(c - 2)
        finish(NC - 2)
        finish(NC - 1)

        for c in range(NC - NPB, NC):
            rs_ops[c].wait_send()
            out_cps[c].wait()
            ag_ops[c].wait_send()
        for c in range(NC):
            ag_ops[c].wait_recv()

    return pl.pallas_call(
        body,
        out_shape=jax.ShapeDtypeStruct((D_HALF, F), jnp.bfloat16),
        in_specs=[
            pl.BlockSpec(memory_space=pl.ANY),
            pl.BlockSpec(memory_space=pl.ANY),
        ],
        out_specs=pl.BlockSpec(memory_space=pl.ANY),
        scratch_shapes=[
            pltpu.VMEM((M, D), jnp.bfloat16),
            pltpu.VMEM((2, XT, D), jnp.float32),
            pltpu.VMEM((3, M, FC), jnp.float32),
            pltpu.VMEM((NPB, D, FC), jnp.bfloat16),
            pltpu.VMEM((NPB, D_HALF, FC), jnp.bfloat16),
            pltpu.VMEM((NC, D_HALF, FC), jnp.bfloat16),
            pltpu.SemaphoreType.DMA((2,)),
            pltpu.SemaphoreType.DMA((3,)),
            pltpu.SemaphoreType.DMA((NPB,)),
            pltpu.SemaphoreType.DMA((NC,)),
            pltpu.SemaphoreType.DMA((NC,)),
            pltpu.SemaphoreType.DMA((NC,)),
            pltpu.SemaphoreType.DMA((NC,)),
        ],
        compiler_params=pltpu.CompilerParams(
            collective_id=0,
            vmem_limit_bytes=60 * 1024 * 1024,
        ),
    )(x, dy)
